# Initial kernel scaffold; baseline (speedup 1.0000x reference)
import jax
import jax.numpy as jnp
from jax.experimental import pallas as pl


def kernel(x, ln_w, ln_b, lin_w, lin_b):
    raise NotImplementedError("write your pallas kernel here")


# speedup vs baseline: 1.1197x; 1.1197x over previous
import functools

import jax
import jax.numpy as jnp
from jax import lax
from jax.experimental import pallas as pl
from jax.experimental.pallas import tpu as pltpu

_B, _N, _D = 4, 4096, 1024
_KEEP = 0.7
_K = int(_N * _KEEP)
_TOTAL = _B * _K
_BN = _B * _N
_EPS = 1e-5
_BIG = 0x7FFFFFF
_JC = 256


def _rank_body(s_ref, dst_ref):
    b = pl.program_id(0)
    s_row = s_ref[0]
    idx_row = lax.broadcasted_iota(jnp.int32, (1, _N), 1)
    acc = jnp.zeros((1, _N), jnp.float32)
    for j0 in range(0, _N, _JC):
        sj = s_row[0:1, j0:j0 + _JC].reshape(_JC, 1)
        jidx = j0 + lax.broadcasted_iota(jnp.int32, (_JC, 1), 0)
        gt = sj > s_row
        eqlt = (sj == s_row) & (jidx < idx_row)
        acc = acc + jnp.sum((gt | eqlt).astype(jnp.float32), axis=0,
                            keepdims=True)
    rank = acc.astype(jnp.int32)
    dst_ref[0] = jnp.where(rank < _K, rank + b * _K, _BIG)


def _ranks(scores):
    out = pl.pallas_call(
        _rank_body,
        grid=(_B,),
        in_specs=[pl.BlockSpec((1, 1, _N), lambda i: (i, 0, 0))],
        out_specs=pl.BlockSpec((1, 1, _N), lambda i: (i, 0, 0)),
        out_shape=jax.ShapeDtypeStruct((_B, 1, _N), jnp.int32),
    )(scores.reshape(_B, 1, _N))
    return out.reshape(_B, _N)


def kernel(x, ln_w, ln_b, lin_w, lin_b):
    mean = jnp.mean(x, axis=-1, keepdims=True)
    var = jnp.var(x, axis=-1, keepdims=True)
    xn = (x - mean) / jnp.sqrt(var + _EPS)
    xn = xn * ln_w + ln_b
    scores = (xn @ lin_w.T + lin_b)[..., 0]

    dst = _ranks(scores).reshape(_BN)

    src = jnp.arange(_BN, dtype=jnp.int32)
    inv = jnp.zeros((_TOTAL,), jnp.int32).at[dst].set(src, mode="drop")
    x_flat = x.reshape(_BN, _D)
    return x_flat[inv].reshape(_B, _K, _D)


# speedup vs baseline: 1.2478x; 1.1144x over previous
import functools

import jax
import jax.numpy as jnp
from jax import lax
from jax.experimental import pallas as pl
from jax.experimental.pallas import tpu as pltpu
from jax.experimental.pallas import tpu_sc as plsc

_B, _N, _D = 4, 4096, 1024
_KEEP = 0.7
_K = int(_N * _KEEP)
_TOTAL = _B * _K
_BN = _B * _N
_EPS = 1e-5
_JC = 256

_NW = 32
_G = 32
_NCH = -(-_TOTAL // _G)
_SLOTS = -(-_NCH // _NW)
_TAIL = _TOTAL - (_NCH - 1) * _G
_PAD = _NCH * _G


def _rank_body(s_ref, src_ref):
    b = pl.program_id(0)
    s_row = s_ref[0]
    idx_row = lax.broadcasted_iota(jnp.int32, (1, _N), 1)
    acc = jnp.zeros((1, _N), jnp.float32)
    for j0 in range(0, _N, _JC):
        sj = s_row[0:1, j0:j0 + _JC].reshape(_JC, 1)
        jidx = j0 + lax.broadcasted_iota(jnp.int32, (_JC, 1), 0)
        gt = sj > s_row
        eqlt = (sj == s_row) & (jidx < idx_row)
        acc = acc + jnp.sum((gt | eqlt).astype(jnp.float32), axis=0,
                            keepdims=True)
    p_row = idx_row.astype(jnp.float32)
    src = jnp.zeros((1, _N), jnp.float32)
    for i0 in range(0, _N, _JC):
        ri = acc[0:1, i0:i0 + _JC].reshape(_JC, 1)
        iv = (i0 + lax.broadcasted_iota(jnp.int32, (_JC, 1), 0)
              ).astype(jnp.float32)
        src = src + jnp.sum(jnp.where(ri == p_row, iv, 0.0), axis=0,
                            keepdims=True)
    src_ref[0] = src.astype(jnp.int32) + b * _N


def _inv_perm(scores):
    out = pl.pallas_call(
        _rank_body,
        grid=(_B,),
        in_specs=[pl.BlockSpec((1, 1, _N), lambda i: (i, 0, 0))],
        out_specs=pl.BlockSpec((1, 1, _N), lambda i: (i, 0, 0)),
        out_shape=jax.ShapeDtypeStruct((_B, 1, _N), jnp.int32),
    )(scores.reshape(_B, 1, _N))
    return out.reshape(_B, _N)


def _sc_gather_body(idx_hbm, x_hbm, out_hbm, idx_v, rows_v, sem0, sem1):
    wid = lax.axis_index("s") * 2 + lax.axis_index("c")
    sems = (sem0, sem1)

    def buf(t):
        return rows_v.at[pl.ds((t % 2) * _G, _G)]

    def fire(t):
        c = t * _NW + wid

        @pl.when(c < _NCH)
        def _():
            pltpu.sync_copy(idx_hbm.at[pl.ds(c * _G, _G)],
                            idx_v.at[pl.ds(t * _G, _G)])
            pltpu.make_async_copy(x_hbm.at[idx_v.at[pl.ds(t * _G, _G)]],
                                  buf(t), sems[t % 2]).start()

    def finish(t):
        c = t * _NW + wid

        @pl.when(c < _NCH)
        def _():
            pltpu.make_async_copy(x_hbm.at[idx_v.at[pl.ds(t * _G, _G)]],
                                  buf(t), sems[t % 2]).wait()

            @pl.when(c < _NCH - 1)
            def _():
                pltpu.sync_copy(buf(t), out_hbm.at[pl.ds(c * _G, _G)])

            @pl.when(c == _NCH - 1)
            def _():
                pltpu.sync_copy(
                    rows_v.at[pl.ds((t % 2) * _G, _TAIL)],
                    out_hbm.at[pl.ds((_NCH - 1) * _G, _TAIL)])

    fire(0)
    for t in range(_SLOTS):
        if t + 1 < _SLOTS:
            fire(t + 1)
        finish(t)


@jax.jit
def _sc_gather(idx, x_flat):
    mesh = plsc.VectorSubcoreMesh(core_axis_name="c", subcore_axis_name="s")
    f = functools.partial(
        pl.kernel,
        out_type=jax.ShapeDtypeStruct((_TOTAL, 8, 128), jnp.float32),
        mesh=mesh,
        scratch_types=[
            pltpu.VMEM((_SLOTS * _G,), jnp.int32),
            pltpu.VMEM((2 * _G, 8, 128), jnp.float32),
            pltpu.SemaphoreType.DMA,
            pltpu.SemaphoreType.DMA,
        ],
    )(_sc_gather_body)
    return f(idx, x_flat)


def kernel(x, ln_w, ln_b, lin_w, lin_b):
    mean = jnp.mean(x, axis=-1, keepdims=True)
    var = jnp.var(x, axis=-1, keepdims=True)
    xn = (x - mean) / jnp.sqrt(var + _EPS)
    xn = xn * ln_w + ln_b
    scores = (xn @ lin_w.T + lin_b)[..., 0]

    src = _inv_perm(scores)
    idx = src[:, :_K].reshape(_TOTAL)
    idx = jnp.pad(idx, (0, _PAD - _TOTAL))
    out = _sc_gather(idx, x.reshape(_BN, 8, 128))
    return out.reshape(_B, _K, _D)


# speedup vs baseline: 1.4833x; 1.1887x over previous
import functools

import jax
import jax.numpy as jnp
from jax import lax
from jax.experimental import pallas as pl
from jax.experimental.pallas import tpu as pltpu
from jax.experimental.pallas import tpu_sc as plsc

_B, _N, _D = 4, 4096, 1024
_KEEP = 0.7
_K = int(_N * _KEEP)
_TOTAL = _B * _K
_BN = _B * _N
_EPS = 1e-5
_JC = 256

_NW = 32
_G = 32
_NCH = _TOTAL // _G
_SLOTS = -(-_NCH // _NW)
_FULL = _NCH * _G


def _rank_body(s_ref, src_ref):
    b = pl.program_id(0)
    s_row = s_ref[0]
    idx_row = lax.broadcasted_iota(jnp.int32, (1, _N), 1)
    acc = jnp.zeros((1, _N), jnp.float32)
    for j0 in range(0, _N, _JC):
        sj = s_row[0:1, j0:j0 + _JC].reshape(_JC, 1)
        jidx = j0 + lax.broadcasted_iota(jnp.int32, (_JC, 1), 0)
        gt = sj > s_row
        eqlt = (sj == s_row) & (jidx < idx_row)
        acc = acc + jnp.sum((gt | eqlt).astype(jnp.float32), axis=0,
                            keepdims=True)
    p_row = idx_row.astype(jnp.float32)
    src = jnp.zeros((1, _N), jnp.float32)
    for i0 in range(0, _N, _JC):
        ri = acc[0:1, i0:i0 + _JC].reshape(_JC, 1)
        iv = (i0 + lax.broadcasted_iota(jnp.int32, (_JC, 1), 0)
              ).astype(jnp.float32)
        src = src + jnp.sum(jnp.where(ri == p_row, iv, 0.0), axis=0,
                            keepdims=True)
    src_ref[0] = src.astype(jnp.int32) + b * _N


def _inv_perm(scores):
    out = pl.pallas_call(
        _rank_body,
        grid=(_B,),
        in_specs=[pl.BlockSpec((1, 1, _N), lambda i: (i, 0, 0))],
        out_specs=pl.BlockSpec((1, 1, _N), lambda i: (i, 0, 0)),
        out_shape=jax.ShapeDtypeStruct((_B, 1, _N), jnp.int32),
    )(scores.reshape(_B, 1, _N))
    return out.reshape(_B, _N)


def _sc_gather_body(idx_hbm, x_hbm, out_hbm, idx_v, rows_v, sem0, sem1):
    wid = lax.axis_index("s") * 2 + lax.axis_index("c")
    sems = (sem0, sem1)

    def buf(t):
        return rows_v.at[t % 2]

    def fire(t):
        c = t * _NW + wid

        @pl.when(c < _NCH)
        def _():
            pltpu.sync_copy(idx_hbm.at[pl.ds(c * _G, _G)],
                            idx_v.at[pl.ds(t * _G, _G)])
            pltpu.make_async_copy(x_hbm.at[idx_v.at[pl.ds(t * _G, _G)]],
                                  buf(t), sems[t % 2]).start()

    def finish(t):
        c = t * _NW + wid

        @pl.when(c < _NCH)
        def _():
            pltpu.make_async_copy(x_hbm.at[idx_v.at[pl.ds(t * _G, _G)]],
                                  buf(t), sems[t % 2]).wait()
            pltpu.sync_copy(buf(t), out_hbm.at[pl.ds(c * _G, _G)])

    fire(0)
    for t in range(_SLOTS):
        if t + 1 < _SLOTS:
            fire(t + 1)
        finish(t)


@jax.jit
def _sc_gather(idx, x_flat):
    mesh = plsc.VectorSubcoreMesh(core_axis_name="c", subcore_axis_name="s")
    f = functools.partial(
        pl.kernel,
        out_type=jax.ShapeDtypeStruct((_TOTAL, _D), jnp.float32),
        mesh=mesh,
        scratch_types=[
            pltpu.VMEM((_SLOTS * _G,), jnp.int32),
            pltpu.VMEM((2, _G, _D), jnp.float32),
            pltpu.SemaphoreType.DMA,
            pltpu.SemaphoreType.DMA,
        ],
    )(_sc_gather_body)
    return f(idx, x_flat)


def kernel(x, ln_w, ln_b, lin_w, lin_b):
    mean = jnp.mean(x, axis=-1, keepdims=True)
    var = jnp.var(x, axis=-1, keepdims=True)
    xn = (x - mean) / jnp.sqrt(var + _EPS)
    xn = xn * ln_w + ln_b
    scores = (xn @ lin_w.T + lin_b)[..., 0]

    src = _inv_perm(scores)
    idx = src[:, :_K].reshape(_TOTAL)
    x_flat = x.reshape(_BN, _D)
    out = _sc_gather(idx[:_FULL], x_flat)
    tail = x_flat[idx[_FULL:]]
    out = lax.dynamic_update_slice(out, tail, (_FULL, 0))
    return out.reshape(_B, _K, _D)
